# baseline (device time: 37802 ns/iter reference)
import jax
import jax.numpy as jnp
from jax import lax
from jax.experimental import pallas as pl
from jax.experimental.pallas import tpu as pltpu

N_DEV = 16
B, SQ, D = 2, 256, 768
DH = 64
HQ_LOC = 8
HKV_LOC = 2
ROWS = B * SQ
CHUNK = ROWS // N_DEV


def kernel(x, Wq, Wo, Wk, Wv):
    x2 = x.reshape(ROWS, D)
    Wk3 = Wk.reshape(D, N_DEV, HKV_LOC * DH)
    Wv3 = Wv.reshape(D, N_DEV, HKV_LOC * DH)

    def body(x_ref, wq_ref, wo_ref, wk_ref, wv_ref, out_ref,
             attn_ref, acc_ref, sstage_ref, stage_ref, g_ref,
             rs_send, rs_recv, ag_send, ag_recv):
        my_i = lax.axis_index("i")

        wq = wq_ref[...].astype(jnp.bfloat16)
        wo = wo_ref[...].astype(jnp.bfloat16)
        wk_loc = wk_ref[:, my_i, :].astype(jnp.bfloat16)
        wv_loc = wv_ref[:, my_i, :].astype(jnp.bfloat16)

        def compute_batch(b):
            r0, r1 = b * SQ, (b + 1) * SQ
            xb = x_ref[r0:r1, :].astype(jnp.bfloat16)
            q = jnp.dot(xb, wq,
                        preferred_element_type=jnp.float32).astype(jnp.bfloat16)
            k = jnp.dot(xb, wk_loc,
                        preferred_element_type=jnp.float32).astype(jnp.bfloat16)
            v = jnp.dot(xb, wv_loc,
                        preferred_element_type=jnp.float32).astype(jnp.bfloat16)
            for g in range(HKV_LOC):
                qs = jnp.concatenate(
                    [q[:, (4 * g + hh) * DH:(4 * g + hh + 1) * DH]
                     for hh in range(4)], axis=0)
                kg = k[:, g * DH:(g + 1) * DH]
                vg = v[:, g * DH:(g + 1) * DH]
                s = lax.dot_general(
                    qs, kg, (((1,), (1,)), ((), ())),
                    preferred_element_type=jnp.float32) * 0.125
                m = jnp.max(s, axis=1, keepdims=True)
                p = jnp.exp(s - m)
                l = jnp.sum(p, axis=1, keepdims=True)
                o = jnp.dot(p.astype(jnp.bfloat16), vg,
                            preferred_element_type=jnp.float32) / l
                for hh in range(4):
                    attn_ref[r0:r1, (4 * g + hh) * DH:(4 * g + hh + 1) * DH] = (
                        o[hh * SQ:(hh + 1) * SQ, :])
            acc_ref[r0:r1, :] = jnp.dot(
                attn_ref[r0:r1, :].astype(jnp.bfloat16), wo,
                preferred_element_type=jnp.float32)

        def rs_rdma(k):
            dest = (my_i + k) % N_DEV
            return pltpu.make_async_remote_copy(
                src_ref=sstage_ref.at[pl.ds(dest * CHUNK, CHUNK), :],
                dst_ref=stage_ref.at[pl.ds(my_i * CHUNK, CHUNK), :],
                send_sem=rs_send.at[k],
                recv_sem=rs_recv.at[k],
                device_id=(dest,),
                device_id_type=pl.DeviceIdType.MESH,
            )

        hi_first = (my_i & 1) == 0

        def compute_and_send(batch):
            r0 = batch * SQ
            sstage_ref[r0:r0 + SQ, :] = acc_ref[
                r0:r0 + SQ, :].astype(jnp.bfloat16)

        @pl.when(hi_first)
        def _():
            compute_batch(1)
            compute_and_send(1)

        @pl.when(jnp.logical_not(hi_first))
        def _():
            compute_batch(0)
            compute_and_send(0)

        barrier = pltpu.get_barrier_semaphore()
        for k in range(1, N_DEV):
            pl.semaphore_signal(barrier, inc=1,
                                device_id=((my_i + k) % N_DEV,),
                                device_id_type=pl.DeviceIdType.MESH)
        pl.semaphore_wait(barrier, N_DEV - 1)

        for k in range(1, N_DEV):
            dest = (my_i + k) % N_DEV

            @pl.when((dest >= N_DEV // 2) == hi_first)
            def _():
                rs_rdma(k).start()

        @pl.when(hi_first)
        def _():
            compute_batch(0)
            compute_and_send(0)

        @pl.when(jnp.logical_not(hi_first))
        def _():
            compute_batch(1)
            compute_and_send(1)

        for k in range(1, N_DEV):
            dest = (my_i + k) % N_DEV

            @pl.when((dest < N_DEV // 2) == hi_first)
            def _():
                rs_rdma(k).start()

        red = acc_ref[pl.ds(my_i * CHUNK, CHUNK), :]
        for k in range(1, N_DEV):
            rs_rdma(k).wait_recv()
            sender = (my_i + N_DEV - k) % N_DEV
            red = red + stage_ref[pl.ds(sender * CHUNK, CHUNK), :].astype(
                jnp.float32)

        g_ref[pl.ds(my_i * CHUNK, CHUNK), :] = red.astype(jnp.bfloat16)

        def ag_rdma(k):
            return pltpu.make_async_remote_copy(
                src_ref=g_ref.at[pl.ds(my_i * CHUNK, CHUNK), :],
                dst_ref=g_ref.at[pl.ds(my_i * CHUNK, CHUNK), :],
                send_sem=ag_send.at[k],
                recv_sem=ag_recv.at[k],
                device_id=((my_i + k) % N_DEV,),
                device_id_type=pl.DeviceIdType.MESH,
            )

        for k in range(1, N_DEV):
            ag_rdma(k).start()
        for k in range(1, N_DEV):
            ag_rdma(k).wait_recv()
        out_ref[...] = g_ref[...].astype(jnp.float32)

        for k in range(1, N_DEV):
            rs_rdma(k).wait_send()
            ag_rdma(k).wait_send()

    out = pl.pallas_call(
        body,
        out_shape=jax.ShapeDtypeStruct((ROWS, D), jnp.float32),
        in_specs=[pl.BlockSpec(memory_space=pltpu.VMEM)] * 5,
        out_specs=pl.BlockSpec(memory_space=pltpu.VMEM),
        scratch_shapes=[
            pltpu.VMEM((ROWS, HQ_LOC * DH), jnp.float32),
            pltpu.VMEM((ROWS, D), jnp.float32),
            pltpu.VMEM((ROWS, D), jnp.bfloat16),
            pltpu.VMEM((ROWS, D), jnp.bfloat16),
            pltpu.VMEM((ROWS, D), jnp.bfloat16),
            pltpu.SemaphoreType.DMA((N_DEV,)),
            pltpu.SemaphoreType.DMA((N_DEV,)),
            pltpu.SemaphoreType.DMA((N_DEV,)),
            pltpu.SemaphoreType.DMA((N_DEV,)),
        ],
        compiler_params=pltpu.CompilerParams(collective_id=0),
    )(x2, Wq, Wo, Wk3, Wv3)
    return out.reshape(B, SQ, D)
